# Initial kernel scaffold; baseline (speedup 1.0000x reference)
import jax
import jax.numpy as jnp
from jax.experimental import pallas as pl


def kernel(feat, edge_index, W0, b0, Wih, Whh, bih, bhh):
    raise NotImplementedError("write your pallas kernel here")


# speedup vs baseline: 6.1005x; 6.1005x over previous
import functools

import jax
import jax.numpy as jnp
from jax import lax
from jax.experimental import pallas as pl
from jax.experimental.pallas import tpu as pltpu
from jax.experimental.pallas import tpu_sc as plsc

STEPS = 3


def _row_block(n):
    for r in (1024, 1000, 512, 500, 256, 250, 200, 128, 125, 80, 40, 8, 1):
        if n % r == 0:
            return r
    return 1


def _linear_body(h_ref, w_ref, b_ref, o_ref):
    o_ref[...] = (
        jnp.dot(h_ref[...], w_ref[...], preferred_element_type=jnp.float32)
        + b_ref[...]
    )


def _linear(h, w_t, b_row):
    n, d = h.shape
    dout = w_t.shape[1]
    r = _row_block(n)
    return pl.pallas_call(
        _linear_body,
        grid=(n // r,),
        in_specs=[
            pl.BlockSpec((r, d), lambda i: (i, 0)),
            pl.BlockSpec((d, dout), lambda i: (0, 0)),
            pl.BlockSpec((1, dout), lambda i: (0, 0)),
        ],
        out_specs=pl.BlockSpec((r, dout), lambda i: (i, 0)),
        out_shape=jax.ShapeDtypeStruct((n, dout), jnp.float32),
    )(h, w_t, b_row)


def _gru_body(a_ref, h_ref, wih_ref, whh_ref, bih_ref, bhh_ref, o_ref):
    a = a_ref[0] + a_ref[1]
    h = h_ref[...]
    d = h.shape[1]
    gi = jnp.dot(a, wih_ref[...], preferred_element_type=jnp.float32) + bih_ref[...]
    gh = jnp.dot(h, whh_ref[...], preferred_element_type=jnp.float32) + bhh_ref[...]
    r = jax.nn.sigmoid(gi[:, :d] + gh[:, :d])
    z = jax.nn.sigmoid(gi[:, d:2 * d] + gh[:, d:2 * d])
    n = jnp.tanh(gi[:, 2 * d:] + r * gh[:, 2 * d:])
    o_ref[...] = (1.0 - z) * n + z * h


def _gru(a_parts, h, wih_t, whh_t, bih_row, bhh_row):
    n, d = h.shape
    r = _row_block(n)
    return pl.pallas_call(
        _gru_body,
        grid=(n // r,),
        in_specs=[
            pl.BlockSpec((2, r, d), lambda i: (0, i, 0)),
            pl.BlockSpec((r, d), lambda i: (i, 0)),
            pl.BlockSpec((d, 3 * d), lambda i: (0, 0)),
            pl.BlockSpec((d, 3 * d), lambda i: (0, 0)),
            pl.BlockSpec((1, 3 * d), lambda i: (0, 0)),
            pl.BlockSpec((1, 3 * d), lambda i: (0, 0)),
        ],
        out_specs=pl.BlockSpec((r, d), lambda i: (i, 0)),
        out_shape=jax.ShapeDtypeStruct((n, d), jnp.float32),
    )(a_parts, h, wih_t, whh_t, bih_row, bhh_row)


NC = 2
NS = 16
NW = NC * NS
CH = 128


@functools.lru_cache(maxsize=None)
def _make_sc_scatter(n, d, e):
    epw = e // NW
    n_full = epw // CH
    rem = epw % CH
    zr = 8
    for c in (128, 80, 64, 40, 32, 16, 8):
        if n % c == 0:
            zr = c
            break
    nchunks = n // zr
    per_sub = -(-nchunks // NS)

    mesh = plsc.VectorSubcoreMesh(core_axis_name="c", subcore_axis_name="s")
    scratch = [
        pltpu.VMEM((CH,), jnp.int32),
        pltpu.VMEM((CH,), jnp.int32),
        pltpu.VMEM((CH, d), jnp.float32),
        pltpu.VMEM_SHARED((n, d), jnp.float32),
        pltpu.VMEM((zr, d), jnp.float32),
        pltpu.SemaphoreType.DMA,
    ]
    if rem:
        scratch += [
            pltpu.VMEM((rem,), jnp.int32),
            pltpu.VMEM((rem,), jnp.int32),
            pltpu.VMEM((rem, d), jnp.float32),
        ]

    def body(m_hbm, src_hbm, dst_hbm, out_hbm, sidx, didx, rows, acc, zbuf,
             sem, *rem_bufs):
        cid = lax.axis_index("c")
        sid = lax.axis_index("s")
        wid = cid * NS + sid

        zero16 = jnp.zeros((16,), jnp.float32)

        def zrow(i, _):
            for c in range(d // 16):
                zbuf[i, pl.ds(c * 16, 16)] = zero16
            return 0

        lax.fori_loop(0, zr, zrow, 0)

        def zchunk(t, _):
            idx = sid + t * NS

            @pl.when(idx < nchunks)
            def _():
                pltpu.sync_copy(zbuf, acc.at[pl.ds(idx * zr, zr)])

            return 0

        lax.fori_loop(0, per_sub, zchunk, 0)
        plsc.subcore_barrier()

        ebase = wid * epw

        def chunk(t, _):
            off = ebase + t * CH
            pltpu.sync_copy(src_hbm.at[pl.ds(off, CH)], sidx)
            pltpu.sync_copy(dst_hbm.at[pl.ds(off, CH)], didx)
            pltpu.async_copy(m_hbm.at[sidx], rows, sem).wait()
            pltpu.sync_copy(rows, acc.at[didx], add=True)
            return 0

        lax.fori_loop(0, n_full, chunk, 0)

        if rem:
            sidx2, didx2, rows2 = rem_bufs
            off = ebase + n_full * CH
            pltpu.sync_copy(src_hbm.at[pl.ds(off, rem)], sidx2)
            pltpu.sync_copy(dst_hbm.at[pl.ds(off, rem)], didx2)
            pltpu.async_copy(m_hbm.at[sidx2], rows2, sem).wait()
            pltpu.sync_copy(rows2, acc.at[didx2], add=True)

        plsc.subcore_barrier()

        def wchunk(t, _):
            idx = sid + t * NS

            @pl.when(idx < nchunks)
            def _():
                sl = pl.ds(idx * zr, zr)
                pltpu.sync_copy(acc.at[sl], out_hbm.at[cid].at[sl])

            return 0

        lax.fori_loop(0, per_sub, wchunk, 0)

    return pl.kernel(
        body,
        out_type=jax.ShapeDtypeStruct((NC, n, d), jnp.float32),
        mesh=mesh,
        scratch_types=scratch,
    )


def _sc_scatter(m, src, dst):
    n, d = m.shape
    return _make_sc_scatter(n, d, src.shape[0])(m, src, dst)


def kernel(feat, edge_index, W0, b0, Wih, Whh, bih, bhh):
    src = edge_index[0]
    dst = edge_index[1]
    w0_t = W0.T
    wih_t = Wih.T
    whh_t = Whh.T
    b0_row = b0.reshape(1, -1)
    bih_row = bih.reshape(1, -1)
    bhh_row = bhh.reshape(1, -1)

    h = feat
    for _ in range(STEPS):
        m = _linear(h, w0_t, b0_row)
        parts = _sc_scatter(m, src, dst)
        h = _gru(parts, h, wih_t, whh_t, bih_row, bhh_row)
    return h
